# Initial kernel scaffold; baseline (speedup 1.0000x reference)
import jax
import jax.numpy as jnp
from jax.experimental import pallas as pl


def kernel(i_goods_id, i_shop_id, i_cate_id, visited_goods_ids, visited_shop_ids, visited_cate_ids, emb_table, W1, b1, g1, be1, a1, W2, b2, g2, be2, a2, W3, b3):
    raise NotImplementedError("write your pallas kernel here")


# speedup vs baseline: 1.3020x; 1.3020x over previous
import functools

import jax
import jax.numpy as jnp
from jax import lax
from jax.experimental import pallas as pl
from jax.experimental.pallas import tpu as pltpu
from jax.experimental.pallas import tpu_sc as plsc

_NC, _NS = 2, 16
_NW = _NC * _NS


def _sc_gather(table, sidx, iidx, E):
    n_ser = sidx.shape[0]
    n_itm = iidx.shape[0]
    per_w_ser = n_ser // _NW
    per_w_itm = n_itm // _NW
    assert n_ser % _NW == 0 and n_itm % _NW == 0
    ch = 2400
    assert per_w_ser % ch == 0 and ch % 8 == 0 and per_w_itm % 8 == 0
    n_ch = per_w_ser // ch

    mesh = plsc.VectorSubcoreMesh(core_axis_name="c", subcore_axis_name="s",
                                  num_cores=_NC, num_subcores=_NS)

    @functools.partial(
        pl.kernel,
        mesh=mesh,
        compiler_params=pltpu.CompilerParams(use_tc_tiling_on_sc=False),
        out_type=(
            jax.ShapeDtypeStruct((n_ser, E), jnp.float32),
            jax.ShapeDtypeStruct((n_itm, E), jnp.float32),
        ),
        scratch_types=[
            pltpu.VMEM((ch,), jnp.int32),
            pltpu.VMEM((ch, E), jnp.float32),
            pltpu.VMEM((per_w_itm,), jnp.int32),
            pltpu.VMEM((per_w_itm, E), jnp.float32),
            pltpu.SemaphoreType.DMA,
        ],
    )
    def gather_k(table_hbm, sidx_hbm, iidx_hbm, out_ser, out_itm,
                 idx_v, rows_v, iidx_v, irows_v, sem):
        wid = lax.axis_index("s") * _NC + lax.axis_index("c")
        ib = wid * per_w_itm
        pltpu.sync_copy(iidx_hbm.at[pl.ds(ib, per_w_itm)], iidx_v)
        pltpu.async_copy(table_hbm.at[iidx_v], irows_v, sem).wait()
        pltpu.sync_copy(irows_v, out_itm.at[pl.ds(ib, per_w_itm)])
        base = wid * per_w_ser

        def step(i, carry):
            off = base + i * ch
            pltpu.sync_copy(sidx_hbm.at[pl.ds(off, ch)], idx_v)
            pltpu.async_copy(table_hbm.at[idx_v], rows_v, sem).wait()
            pltpu.sync_copy(rows_v, out_ser.at[pl.ds(off, ch)])
            return carry

        lax.fori_loop(0, n_ch, step, 0)

    return gather_k(table, sidx, iidx)


def _attention(xs, xi, vg, bblk):
    B, L, D = xs.shape
    grid = B // bblk

    def att_body(xs_ref, xi_ref, vg_ref, pooled_ref, mask_ref):
        x = xs_ref[...]
        item = xi_ref[...]
        mask = vg_ref[...] != 0
        scores = jnp.sum(x * item[:, None, :], axis=2)
        ms = scores * mask.astype(jnp.float32)
        pooled_ref[...] = jnp.sum(x * ms[:, :, None], axis=1)
        mask_ref[...] = mask

    return pl.pallas_call(
        att_body,
        grid=(grid,),
        in_specs=[
            pl.BlockSpec((bblk, L, D), lambda i: (i, 0, 0)),
            pl.BlockSpec((bblk, D), lambda i: (i, 0)),
            pl.BlockSpec((bblk, L), lambda i: (i, 0)),
        ],
        out_specs=[
            pl.BlockSpec((bblk, D), lambda i: (i, 0)),
            pl.BlockSpec((bblk, L), lambda i: (i, 0)),
        ],
        out_shape=[
            jax.ShapeDtypeStruct((B, D), jnp.float32),
            jax.ShapeDtypeStruct((B, L), jnp.bool_),
        ],
    )(xs, xi, vg)


def _mlp(xi, pooled, W1, b1, g1, be1, a1, W2, b2, g2, be2, a2, W3, b3):
    B = xi.shape[0]
    OUT = W3.shape[1]
    eps = 1e-3

    def layernorm(h, g, be):
        mu = jnp.mean(h, axis=-1, keepdims=True)
        var = jnp.mean((h - mu) ** 2, axis=-1, keepdims=True)
        return g * (h - mu) / jnp.sqrt(var + eps) + be

    def dice(h, alpha):
        mu = jnp.mean(h, axis=0, keepdims=True)
        var = jnp.mean((h - mu) ** 2, axis=0, keepdims=True)
        xn = (h - mu) / jnp.sqrt(var + eps)
        p = 1.0 / (1.0 + jnp.exp(-xn))
        return alpha * (1.0 - p) * h + p * h

    def mlp_body(xi_ref, pl_ref, W1_ref, b1_ref, g1_ref, be1_ref, a1_ref,
                 W2_ref, b2_ref, g2_ref, be2_ref, a2_ref, W3_ref, b3_ref,
                 out_ref):
        X = jnp.concatenate([xi_ref[...], pl_ref[...]], axis=1)
        h = jnp.dot(X, W1_ref[...], preferred_element_type=jnp.float32) + b1_ref[...]
        h = layernorm(h, g1_ref[...], be1_ref[...])
        h = dice(h, a1_ref[...])
        h = jnp.dot(h, W2_ref[...], preferred_element_type=jnp.float32) + b2_ref[...]
        h = layernorm(h, g2_ref[...], be2_ref[...])
        h = dice(h, a2_ref[...])
        logits = jnp.dot(h, W3_ref[...], preferred_element_type=jnp.float32) + b3_ref[...]
        m = jnp.max(logits, axis=-1, keepdims=True)
        e = jnp.exp(logits - m)
        out_ref[...] = e / jnp.sum(e, axis=-1, keepdims=True)

    args = (xi, pooled, W1, b1.reshape(1, -1), g1.reshape(1, -1),
            be1.reshape(1, -1), a1.reshape(1, -1), W2, b2.reshape(1, -1),
            g2.reshape(1, -1), be2.reshape(1, -1), a2.reshape(1, -1), W3,
            b3.reshape(1, -1))
    return pl.pallas_call(
        mlp_body,
        out_shape=jax.ShapeDtypeStruct((B, OUT), jnp.float32),
    )(*args)


def kernel(i_goods_id, i_shop_id, i_cate_id, visited_goods_ids,
           visited_shop_ids, visited_cate_ids, emb_table, W1, b1, g1, be1, a1,
           W2, b2, g2, be2, a2, W3, b3):
    B, L = visited_goods_ids.shape
    E = emb_table.shape[1]
    sidx = jnp.stack([visited_goods_ids, visited_shop_ids, visited_cate_ids],
                     axis=2).reshape(B * L * 3)
    iidx = jnp.stack([i_goods_id, i_shop_id, i_cate_id], axis=1).reshape(B * 3)

    rows_ser, rows_itm = _sc_gather(emb_table, sidx, iidx, E)
    X_series = rows_ser.reshape(B, L, 3 * E)
    X_item = rows_itm.reshape(B, 3 * E)

    pooled, mask = _attention(X_series, X_item, visited_goods_ids, 128)
    output = _mlp(X_item, pooled, W1, b1, g1, be1, a1, W2, b2, g2, be2, a2,
                  W3, b3)
    return output, X_series, mask


# speedup vs baseline: 3.6769x; 2.8240x over previous
import functools

import jax
import jax.numpy as jnp
from jax import lax
from jax.experimental import pallas as pl
from jax.experimental.pallas import tpu as pltpu
from jax.experimental.pallas import tpu_sc as plsc

_NC, _NS = 2, 16
_NW = _NC * _NS


def _sc_gather(table, vg_f, vs_f, vc_f, iidx, E):
    n_pl = vg_f.shape[0]
    n_itm = iidx.shape[0]
    per_w_pl = n_pl // _NW
    per_w_itm = n_itm // _NW
    assert n_pl % _NW == 0 and n_itm % _NW == 0
    chp = 800
    ch = 3 * chp
    assert per_w_pl % chp == 0 and chp % 16 == 0 and per_w_itm % 8 == 0
    n_ch = per_w_pl // chp

    mesh = plsc.VectorSubcoreMesh(core_axis_name="c", subcore_axis_name="s",
                                  num_cores=_NC, num_subcores=_NS)

    @functools.partial(
        pl.kernel,
        mesh=mesh,
        compiler_params=pltpu.CompilerParams(use_tc_tiling_on_sc=False,
                                             needs_layout_passes=False),
        out_type=(
            jax.ShapeDtypeStruct((3 * n_pl, E), jnp.float32),
            jax.ShapeDtypeStruct((n_itm, E), jnp.float32),
        ),
        scratch_types=[
            pltpu.VMEM((chp,), jnp.int32),
            pltpu.VMEM((chp,), jnp.int32),
            pltpu.VMEM((chp,), jnp.int32),
            pltpu.VMEM((ch,), jnp.int32),
            pltpu.VMEM((ch, E), jnp.float32),
            pltpu.VMEM((per_w_itm,), jnp.int32),
            pltpu.VMEM((per_w_itm, E), jnp.float32),
            pltpu.SemaphoreType.DMA,
        ],
    )
    def gather_k(table_hbm, vg_hbm, vs_hbm, vc_hbm, iidx_hbm, out_ser, out_itm,
                 g_v, s_v, c_v, idx_v, rows_v, iidx_v, irows_v, sem):
        wid = lax.axis_index("s") * _NC + lax.axis_index("c")
        ib = wid * per_w_itm
        pltpu.sync_copy(iidx_hbm.at[pl.ds(ib, per_w_itm)], iidx_v)
        pltpu.async_copy(table_hbm.at[iidx_v], irows_v, sem).wait()
        pltpu.sync_copy(irows_v, out_itm.at[pl.ds(ib, per_w_itm)])
        base = wid * per_w_pl
        lanes3 = 3 * lax.iota(jnp.int32, 16)

        def step(i, carry):
            p0 = base + i * chp
            pltpu.sync_copy(vg_hbm.at[pl.ds(p0, chp)], g_v)
            pltpu.sync_copy(vs_hbm.at[pl.ds(p0, chp)], s_v)
            pltpu.sync_copy(vc_hbm.at[pl.ds(p0, chp)], c_v)

            def grp(k, c2):
                pos = lanes3 + 3 * (k * 16)
                plsc.store_scatter(idx_v, [pos], g_v[pl.ds(k * 16, 16)])
                plsc.store_scatter(idx_v, [pos + 1], s_v[pl.ds(k * 16, 16)])
                plsc.store_scatter(idx_v, [pos + 2], c_v[pl.ds(k * 16, 16)])
                return c2

            lax.fori_loop(0, chp // 16, grp, 0)
            pltpu.async_copy(table_hbm.at[idx_v], rows_v, sem).wait()
            pltpu.sync_copy(rows_v, out_ser.at[pl.ds(3 * p0, ch)])
            return carry

        lax.fori_loop(0, n_ch, step, 0)

    return gather_k(table, vg_f, vs_f, vc_f, iidx)


def _attention(xs, xi, vg, bblk):
    B, L, D = xs.shape
    grid = B // bblk

    def att_body(xs_ref, xi_ref, vg_ref, pooled_ref, mask_ref):
        x = xs_ref[...]
        item = xi_ref[...]
        mask = vg_ref[...] != 0
        scores = jnp.sum(x * item[:, None, :], axis=2)
        ms = scores * mask.astype(jnp.float32)
        pooled_ref[...] = jnp.sum(x * ms[:, :, None], axis=1)
        mask_ref[...] = mask

    return pl.pallas_call(
        att_body,
        grid=(grid,),
        in_specs=[
            pl.BlockSpec((bblk, L, D), lambda i: (i, 0, 0)),
            pl.BlockSpec((bblk, D), lambda i: (i, 0)),
            pl.BlockSpec((bblk, L), lambda i: (i, 0)),
        ],
        out_specs=[
            pl.BlockSpec((bblk, D), lambda i: (i, 0)),
            pl.BlockSpec((bblk, L), lambda i: (i, 0)),
        ],
        out_shape=[
            jax.ShapeDtypeStruct((B, D), jnp.float32),
            jax.ShapeDtypeStruct((B, L), jnp.bool_),
        ],
    )(xs, xi, vg)


def _mlp(xi, pooled, W1, b1, g1, be1, a1, W2, b2, g2, be2, a2, W3, b3):
    B = xi.shape[0]
    OUT = W3.shape[1]
    eps = 1e-3

    def layernorm(h, g, be):
        mu = jnp.mean(h, axis=-1, keepdims=True)
        var = jnp.mean((h - mu) ** 2, axis=-1, keepdims=True)
        return g * (h - mu) / jnp.sqrt(var + eps) + be

    def dice(h, alpha):
        mu = jnp.mean(h, axis=0, keepdims=True)
        var = jnp.mean((h - mu) ** 2, axis=0, keepdims=True)
        xn = (h - mu) / jnp.sqrt(var + eps)
        p = 1.0 / (1.0 + jnp.exp(-xn))
        return alpha * (1.0 - p) * h + p * h

    def mlp_body(xi_ref, pl_ref, W1_ref, b1_ref, g1_ref, be1_ref, a1_ref,
                 W2_ref, b2_ref, g2_ref, be2_ref, a2_ref, W3_ref, b3_ref,
                 out_ref):
        X = jnp.concatenate([xi_ref[...], pl_ref[...]], axis=1)
        h = jnp.dot(X, W1_ref[...], preferred_element_type=jnp.float32) + b1_ref[...]
        h = layernorm(h, g1_ref[...], be1_ref[...])
        h = dice(h, a1_ref[...])
        h = jnp.dot(h, W2_ref[...], preferred_element_type=jnp.float32) + b2_ref[...]
        h = layernorm(h, g2_ref[...], be2_ref[...])
        h = dice(h, a2_ref[...])
        logits = jnp.dot(h, W3_ref[...], preferred_element_type=jnp.float32) + b3_ref[...]
        m = jnp.max(logits, axis=-1, keepdims=True)
        e = jnp.exp(logits - m)
        out_ref[...] = e / jnp.sum(e, axis=-1, keepdims=True)

    args = (xi, pooled, W1, b1.reshape(1, -1), g1.reshape(1, -1),
            be1.reshape(1, -1), a1.reshape(1, -1), W2, b2.reshape(1, -1),
            g2.reshape(1, -1), be2.reshape(1, -1), a2.reshape(1, -1), W3,
            b3.reshape(1, -1))
    return pl.pallas_call(
        mlp_body,
        out_shape=jax.ShapeDtypeStruct((B, OUT), jnp.float32),
    )(*args)


def kernel(i_goods_id, i_shop_id, i_cate_id, visited_goods_ids,
           visited_shop_ids, visited_cate_ids, emb_table, W1, b1, g1, be1, a1,
           W2, b2, g2, be2, a2, W3, b3):
    B, L = visited_goods_ids.shape
    E = emb_table.shape[1]
    iidx = jnp.stack([i_goods_id, i_shop_id, i_cate_id], axis=1).reshape(B * 3)

    rows_ser, rows_itm = _sc_gather(emb_table, visited_goods_ids.reshape(-1),
                                    visited_shop_ids.reshape(-1),
                                    visited_cate_ids.reshape(-1), iidx, E)
    X_series = rows_ser.reshape(B, L, 3 * E)
    X_item = rows_itm.reshape(B, 3 * E)

    pooled, mask = _attention(X_series, X_item, visited_goods_ids, 128)
    output = _mlp(X_item, pooled, W1, b1, g1, be1, a1, W2, b2, g2, be2, a2,
                  W3, b3)
    return output, X_series, mask


# speedup vs baseline: 5.2339x; 1.4234x over previous
import functools

import jax
import jax.numpy as jnp
from jax import lax
from jax.experimental import pallas as pl
from jax.experimental.pallas import tpu as pltpu
from jax.experimental.pallas import tpu_sc as plsc

_NC, _NS = 2, 16
_NW = _NC * _NS


def _sc_gather(table, vg_f, vs_f, vc_f, iidx, E):
    n_pl = vg_f.shape[0]
    n_itm = iidx.shape[0]
    per_w_pl = n_pl // _NW
    per_w_itm = n_itm // _NW
    assert n_pl % _NW == 0 and n_itm % _NW == 0
    chp = 800
    ch = 3 * chp
    assert per_w_pl % chp == 0 and chp % 16 == 0 and per_w_itm % 8 == 0
    n_ch = per_w_pl // chp

    mesh = plsc.VectorSubcoreMesh(core_axis_name="c", subcore_axis_name="s",
                                  num_cores=_NC, num_subcores=_NS)

    @functools.partial(
        pl.kernel,
        mesh=mesh,
        compiler_params=pltpu.CompilerParams(use_tc_tiling_on_sc=False,
                                             needs_layout_passes=False),
        out_type=(
            jax.ShapeDtypeStruct((3 * n_pl, E), jnp.float32),
            jax.ShapeDtypeStruct((n_itm, E), jnp.float32),
        ),
        scratch_types=[
            pltpu.VMEM((chp,), jnp.int32),
            pltpu.VMEM((chp,), jnp.int32),
            pltpu.VMEM((chp,), jnp.int32),
            pltpu.VMEM((ch,), jnp.int32),
            pltpu.VMEM((ch, E), jnp.float32),
            pltpu.VMEM((per_w_itm,), jnp.int32),
            pltpu.VMEM((per_w_itm, E), jnp.float32),
            pltpu.SemaphoreType.DMA,
        ],
    )
    def gather_k(table_hbm, vg_hbm, vs_hbm, vc_hbm, iidx_hbm, out_ser, out_itm,
                 g_v, s_v, c_v, idx_v, rows_v, iidx_v, irows_v, sem):
        wid = lax.axis_index("s") * _NC + lax.axis_index("c")
        ib = wid * per_w_itm
        pltpu.sync_copy(iidx_hbm.at[pl.ds(ib, per_w_itm)], iidx_v)
        pltpu.async_copy(table_hbm.at[iidx_v], irows_v, sem).wait()
        pltpu.sync_copy(irows_v, out_itm.at[pl.ds(ib, per_w_itm)])
        base = wid * per_w_pl
        lanes3 = 3 * lax.iota(jnp.int32, 16)

        def step(i, carry):
            p0 = base + i * chp
            pltpu.sync_copy(vg_hbm.at[pl.ds(p0, chp)], g_v)
            pltpu.sync_copy(vs_hbm.at[pl.ds(p0, chp)], s_v)
            pltpu.sync_copy(vc_hbm.at[pl.ds(p0, chp)], c_v)

            def grp(k, c2):
                pos = lanes3 + 3 * (k * 16)
                plsc.store_scatter(idx_v, [pos], g_v[pl.ds(k * 16, 16)])
                plsc.store_scatter(idx_v, [pos + 1], s_v[pl.ds(k * 16, 16)])
                plsc.store_scatter(idx_v, [pos + 2], c_v[pl.ds(k * 16, 16)])
                return c2

            lax.fori_loop(0, chp // 16, grp, 0)
            pltpu.async_copy(table_hbm.at[idx_v], rows_v, sem).wait()
            pltpu.sync_copy(rows_v, out_ser.at[pl.ds(3 * p0, ch)])
            return carry

        lax.fori_loop(0, n_ch, step, 0)

    return gather_k(table, vg_f, vs_f, vc_f, iidx)


def _attention_t(xs2, xiT, vgT, B, L, D, bblk):
    grid = B // bblk
    lpb = bblk * L * D // (128 * bblk)
    assert (L * D) % 128 == 0
    nq = L * D // 128

    def att_body(xs_ref, xi_ref, vg_ref, xsT_ref, pooled_ref, mask_ref):
        blk = xs_ref[...].reshape(bblk, nq, 128)
        xsT = jnp.transpose(blk, (1, 2, 0)).reshape(L * D, bblk)
        xsT_ref[...] = xsT
        xsT3 = xsT.reshape(L, D, bblk)
        item = xi_ref[...]
        mask = vg_ref[...] != 0
        scores = jnp.sum(xsT3 * item[None, :, :], axis=1)
        ms = scores * mask.astype(jnp.float32)
        pooled_ref[...] = jnp.sum(xsT3 * ms[:, None, :], axis=0)
        mask_ref[...] = mask

    return pl.pallas_call(
        att_body,
        grid=(grid,),
        in_specs=[
            pl.BlockSpec((nq * bblk, 128), lambda i: (i, 0)),
            pl.BlockSpec((D, bblk), lambda i: (0, i)),
            pl.BlockSpec((L, bblk), lambda i: (0, i)),
        ],
        out_specs=[
            pl.BlockSpec((L * D, bblk), lambda i: (0, i)),
            pl.BlockSpec((D, bblk), lambda i: (0, i)),
            pl.BlockSpec((L, bblk), lambda i: (0, i)),
        ],
        out_shape=[
            jax.ShapeDtypeStruct((L * D, B), jnp.float32),
            jax.ShapeDtypeStruct((D, B), jnp.float32),
            jax.ShapeDtypeStruct((L, B), jnp.bool_),
        ],
    )(xs2, xiT, vgT)


def _mlp_t(xiT, pooledT, W1, b1, g1, be1, a1, W2, b2, g2, be2, a2, W3, b3):
    B = xiT.shape[1]
    OUT = W3.shape[1]
    eps = 1e-3

    def layernorm(h, g, be):
        mu = jnp.mean(h, axis=0, keepdims=True)
        var = jnp.mean((h - mu) ** 2, axis=0, keepdims=True)
        return g * (h - mu) / jnp.sqrt(var + eps) + be

    def dice(h, alpha):
        mu = jnp.mean(h, axis=1, keepdims=True)
        var = jnp.mean((h - mu) ** 2, axis=1, keepdims=True)
        xn = (h - mu) / jnp.sqrt(var + eps)
        p = 1.0 / (1.0 + jnp.exp(-xn))
        return alpha * (1.0 - p) * h + p * h

    def mlp_body(xi_ref, pl_ref, W1_ref, b1_ref, g1_ref, be1_ref, a1_ref,
                 W2_ref, b2_ref, g2_ref, be2_ref, a2_ref, W3_ref, b3_ref,
                 out_ref):
        XT = jnp.concatenate([xi_ref[...], pl_ref[...]], axis=0)
        h = jnp.dot(W1_ref[...], XT, preferred_element_type=jnp.float32) + b1_ref[...]
        h = layernorm(h, g1_ref[...], be1_ref[...])
        h = dice(h, a1_ref[...])
        h = jnp.dot(W2_ref[...], h, preferred_element_type=jnp.float32) + b2_ref[...]
        h = layernorm(h, g2_ref[...], be2_ref[...])
        h = dice(h, a2_ref[...])
        logits = jnp.dot(W3_ref[...], h, preferred_element_type=jnp.float32) + b3_ref[...]
        m = jnp.max(logits, axis=0, keepdims=True)
        e = jnp.exp(logits - m)
        out_ref[...] = e / jnp.sum(e, axis=0, keepdims=True)

    args = (xiT, pooledT, W1.T, b1.reshape(-1, 1), g1.reshape(-1, 1),
            be1.reshape(-1, 1), a1.reshape(-1, 1), W2.T, b2.reshape(-1, 1),
            g2.reshape(-1, 1), be2.reshape(-1, 1), a2.reshape(-1, 1), W3.T,
            b3.reshape(-1, 1))
    return pl.pallas_call(
        mlp_body,
        out_shape=jax.ShapeDtypeStruct((OUT, B), jnp.float32),
    )(*args)


def kernel(i_goods_id, i_shop_id, i_cate_id, visited_goods_ids,
           visited_shop_ids, visited_cate_ids, emb_table, W1, b1, g1, be1, a1,
           W2, b2, g2, be2, a2, W3, b3):
    B, L = visited_goods_ids.shape
    E = emb_table.shape[1]
    iidx = jnp.stack([i_goods_id, i_shop_id, i_cate_id], axis=1).reshape(B * 3)

    rows_ser, rows_itm = _sc_gather(emb_table, visited_goods_ids.reshape(-1),
                                    visited_shop_ids.reshape(-1),
                                    visited_cate_ids.reshape(-1), iidx, E)
    D = 3 * E
    xs2 = rows_ser.reshape(B * L * D // 128, 128)
    xiT = rows_itm.reshape(B, D).T
    vgT = visited_goods_ids.T

    xsT, pooledT, maskT = _attention_t(xs2, xiT, vgT, B, L, D, 128)
    outT = _mlp_t(xiT, pooledT, W1, b1, g1, be1, a1, W2, b2, g2, be2, a2,
                  W3, b3)
    X_series = xsT.reshape(L, D, B).transpose(2, 0, 1)
    return outT.T, X_series, maskT.T
